# Initial kernel scaffold; baseline (speedup 1.0000x reference)
import jax
import jax.numpy as jnp
from jax.experimental import pallas as pl


def kernel(params, weights, t):
    raise NotImplementedError("write your pallas kernel here")


# speedup vs baseline: 53.9958x; 53.9958x over previous
import functools

import jax
import jax.numpy as jnp
from jax import lax
from jax.experimental import pallas as pl
from jax.experimental.pallas import tpu as pltpu
from jax.experimental.pallas import tpu_sc as plsc

KS = 5
D = 32
NC = 2
NS = 16
L = 16
CHUNK = 512


def _make_sc_kernel(n_total, n_rows):
    nw = NC * NS
    per_w = n_total // nw
    n_chunks = per_w // CHUNK
    mesh = plsc.VectorSubcoreMesh(
        core_axis_name="c", subcore_axis_name="s",
        num_cores=NC, num_subcores=NS)

    @functools.partial(
        pl.kernel,
        out_type=jax.ShapeDtypeStruct((n_total, D), jnp.float32),
        mesh=mesh,
        compiler_params=pltpu.CompilerParams(use_tc_tiling_on_sc=False),
        scratch_types=[
            [pltpu.VMEM((CHUNK,), jnp.int32) for _ in range(KS)],
            pltpu.VMEM((KS, CHUNK, D), jnp.float32),
            pltpu.VMEM((KS * L,), jnp.float32),
            pltpu.SemaphoreType.DMA,
        ],
    )
    def sc_kernel(params_hbm, tflat_hbm, wexp_hbm, out_hbm,
                  idx_refs, rows_ref, w_ref, sem):
        wid = lax.axis_index("s") * NC + lax.axis_index("c")
        base = wid * per_w

        pltpu.sync_copy(wexp_hbm, w_ref)
        wvecs = [w_ref[pl.ds(k * L, L)] for k in range(KS)]

        def chunk_body(c, _):
            off = base + c * CHUNK
            pltpu.sync_copy(tflat_hbm.at[pl.ds(off, CHUNK)], idx_refs[0])
            for i in range(CHUNK // L):
                v = idx_refs[0][pl.ds(i * L, L)]
                for k in range(1, KS):
                    idx_refs[k][pl.ds(i * L, L)] = v + k
            copies = [
                pltpu.async_copy(params_hbm.at[idx_refs[k]],
                                 rows_ref.at[k], sem)
                for k in range(KS)
            ]
            for cp in copies:
                cp.wait()

            def combine(r, _):
                for h in (0, L):
                    acc = rows_ref[0, r, pl.ds(h, L)] * wvecs[0]
                    for k in range(1, KS):
                        acc = acc + rows_ref[k, r, pl.ds(h, L)] * wvecs[k]
                    rows_ref[0, r, pl.ds(h, L)] = acc
                return 0

            lax.fori_loop(0, CHUNK, combine, 0)
            pltpu.sync_copy(rows_ref.at[0], out_hbm.at[pl.ds(off, CHUNK)])
            return 0

        lax.fori_loop(0, n_chunks, chunk_body, 0)

    return sc_kernel


def kernel(params, weights, t):
    b, l = t.shape
    n_total = b * l
    tflat = t.reshape(n_total)
    wexp = jnp.repeat(weights, L)
    out = _make_sc_kernel(n_total, params.shape[0])(params, tflat, wexp)
    return out.reshape(b, l, D)


# speedup vs baseline: 56.3283x; 1.0432x over previous
import functools

import jax
import jax.numpy as jnp
from jax import lax
from jax.experimental import pallas as pl
from jax.experimental.pallas import tpu as pltpu
from jax.experimental.pallas import tpu_sc as plsc

KS = 5
D = 32
NC = 2
NS = 16
L = 16
CHUNK = 512


def _make_sc_kernel(n_total, n_rows):
    nw = NC * NS
    per_w = n_total // nw
    n_chunks = per_w // CHUNK
    mesh = plsc.VectorSubcoreMesh(
        core_axis_name="c", subcore_axis_name="s",
        num_cores=NC, num_subcores=NS)

    @functools.partial(
        pl.kernel,
        out_type=jax.ShapeDtypeStruct((n_total, D), jnp.float32),
        mesh=mesh,
        compiler_params=pltpu.CompilerParams(use_tc_tiling_on_sc=False),
        scratch_types=[
            [pltpu.VMEM((CHUNK,), jnp.int32) for _ in range(KS)],
            pltpu.VMEM((KS, CHUNK, D), jnp.float32),
            pltpu.VMEM((KS * L,), jnp.float32),
            pltpu.SemaphoreType.DMA,
        ],
    )
    def sc_kernel(params_hbm, tflat_hbm, wexp_hbm, out_hbm,
                  idx_refs, rows_ref, w_ref, sem):
        wid = lax.axis_index("s") * NC + lax.axis_index("c")
        base = wid * per_w

        pltpu.sync_copy(wexp_hbm, w_ref)
        wvecs = [w_ref[pl.ds(k * L, L)] for k in range(KS)]

        def chunk_body(c, _):
            off = base + c * CHUNK
            pltpu.sync_copy(tflat_hbm.at[pl.ds(off, CHUNK)], idx_refs[0])
            for i in range(CHUNK // L):
                v = idx_refs[0][pl.ds(i * L, L)]
                for k in range(1, KS):
                    idx_refs[k][pl.ds(i * L, L)] = v + k
            copies = [
                pltpu.async_copy(params_hbm.at[idx_refs[k]],
                                 rows_ref.at[k], sem)
                for k in range(KS)
            ]
            for cp in copies:
                cp.wait()

            @plsc.parallel_loop(0, CHUNK, 1, unroll=8)
            def _combine(r):
                for h in (0, L):
                    acc = rows_ref[0, r, pl.ds(h, L)] * wvecs[0]
                    for k in range(1, KS):
                        acc = acc + rows_ref[k, r, pl.ds(h, L)] * wvecs[k]
                    rows_ref[0, r, pl.ds(h, L)] = acc
            pltpu.sync_copy(rows_ref.at[0], out_hbm.at[pl.ds(off, CHUNK)])
            return 0

        lax.fori_loop(0, n_chunks, chunk_body, 0)

    return sc_kernel


def kernel(params, weights, t):
    b, l = t.shape
    n_total = b * l
    tflat = t.reshape(n_total)
    wexp = jnp.repeat(weights, L)
    out = _make_sc_kernel(n_total, params.shape[0])(params, tflat, wexp)
    return out.reshape(b, l, D)
